# baseline (speedup 1.0000x reference)
import functools

import jax
import jax.numpy as jnp
from jax import lax
from jax.experimental import pallas as pl
from jax.experimental.pallas import tpu as pltpu
from jax.experimental.pallas import tpu_sc as plsc

N_ROWS = 1024
N_COLS = 100000
SMOOTHING = 0.1
CONFIDENCE = 1.0 - SMOOTHING

C_TC = 80000
K_SC = 19968
C_TAIL = C_TC + K_SC
TAIL_W = N_COLS - C_TAIL

R_BLK = 128
C_BLK = 16000

NC, NS, L = 2, 16, 16
NW = NC * NS
ROWS_PER_CORE = N_ROWS // NC
N_CHUNKS = ROWS_PER_CORE // NS
FS = 4
RPF = NS // FS
U = 16


def _tc_kernel(x_ref, xtail_ref, se_ref, sx_ref, s_acc, x_acc):
    cb = pl.program_id(1)
    n_cb = pl.num_programs(1)

    @pl.when(cb == 0)
    def _init():
        x = xtail_ref[...]
        lanes = jax.lax.broadcasted_iota(jnp.int32, x.shape, 1)
        valid = lanes < TAIL_W
        e = jnp.exp(jnp.where(valid, jnp.minimum(x, 60.0), -jnp.inf))
        s_acc[...] = jnp.sum(e, axis=1, keepdims=True)
        x_acc[...] = jnp.sum(jnp.where(valid, x, 0.0), axis=1, keepdims=True)

    x = x_ref[...]
    e = jnp.exp(jnp.minimum(x, 60.0))
    s_acc[...] += jnp.sum(e, axis=1, keepdims=True)
    x_acc[...] += jnp.sum(x, axis=1, keepdims=True)

    @pl.when(cb == n_cb - 1)
    def _fin():
        se_ref[...] = s_acc[...]
        sx_ref[...] = x_acc[...]


def _tc_main(inputs):
    n_rb = N_ROWS // R_BLK
    n_cb = C_TC // C_BLK
    return pl.pallas_call(
        _tc_kernel,
        grid=(n_rb, n_cb),
        in_specs=[
            pl.BlockSpec((R_BLK, C_BLK), lambda rb, cb: (rb, cb)),
            pl.BlockSpec((R_BLK, 128), lambda rb, cb: (rb, C_TAIL // 128)),
        ],
        out_specs=[
            pl.BlockSpec((R_BLK, 1), lambda rb, cb: (rb, 0)),
            pl.BlockSpec((R_BLK, 1), lambda rb, cb: (rb, 0)),
        ],
        out_shape=[jax.ShapeDtypeStruct((N_ROWS, 1), jnp.float32),
                   jax.ShapeDtypeStruct((N_ROWS, 1), jnp.float32)],
        scratch_shapes=[
            pltpu.VMEM((R_BLK, 1), jnp.float32),
            pltpu.VMEM((R_BLK, 1), jnp.float32),
        ],
    )(inputs, inputs)


def _tree(vs):
    while len(vs) > 1:
        nxt = [vs[j] + vs[j + 1] for j in range(0, len(vs) - 1, 2)]
        if len(vs) % 2:
            nxt.append(vs[-1])
        vs = nxt
    return vs[0]


def _sc_body(x_hbm, se_hbm, sx_hbm, sp0, sp1, tbuf, st_se, st_sx, semf):
    c = lax.axis_index("c")
    s = lax.axis_index("s")
    zeros = jnp.zeros((L,), jnp.float32)
    sps = (sp0, sp1)

    def fill_desc(g, spbuf):
        row0 = c * ROWS_PER_CORE + g * NS + s * RPF
        return pltpu.make_async_copy(
            x_hbm.at[pl.ds(row0, RPF), pl.ds(C_TC, K_SC)],
            spbuf.at[pl.ds(s * RPF, RPF)], semf)

    @pl.when(s < FS)
    def _prime():
        fill_desc(0, sp0).start()
        fill_desc(0, sp0).wait()

    plsc.subcore_barrier()

    for g in range(N_CHUNKS):
        cur = sps[g % 2]
        nxt = sps[(g + 1) % 2]
        if g + 1 < N_CHUNKS:
            @pl.when(s < FS)
            def _start_next(g=g, nxt=nxt):
                fill_desc(g + 1, nxt).start()

        pltpu.sync_copy(cur.at[s], tbuf)

        def ch(i, cry):
            se, sx = cry
            base = i * (U * L)
            xs = [tbuf[pl.ds(base + u * L, L)] for u in range(U)]
            es = [jnp.exp(jnp.minimum(xv, 60.0)) for xv in xs]
            return se + _tree(es), sx + _tree(xs)

        se16, sx16 = lax.fori_loop(0, K_SC // (U * L), ch, (zeros, zeros))
        st_se[...] = se16
        st_sx[...] = sx16
        row = c * ROWS_PER_CORE + g * NS + s
        pltpu.sync_copy(st_se, se_hbm.at[row])
        pltpu.sync_copy(st_sx, sx_hbm.at[row])

        if g + 1 < N_CHUNKS:
            @pl.when(s < FS)
            def _wait_next(g=g, nxt=nxt):
                fill_desc(g + 1, nxt).wait()

        plsc.subcore_barrier()


def _sc_main(inputs):
    mesh = plsc.VectorSubcoreMesh(core_axis_name="c", subcore_axis_name="s")
    return pl.kernel(
        _sc_body,
        out_type=(jax.ShapeDtypeStruct((N_ROWS, L), jnp.float32),
                  jax.ShapeDtypeStruct((N_ROWS, L), jnp.float32)),
        mesh=mesh,
        scratch_types=[
            pltpu.MemorySpace.VMEM_SHARED((NS, K_SC), jnp.float32),
            pltpu.MemorySpace.VMEM_SHARED((NS, K_SC), jnp.float32),
            pltpu.VMEM((K_SC,), jnp.float32),
            pltpu.VMEM((L,), jnp.float32),
            pltpu.VMEM((L,), jnp.float32),
            pltpu.SemaphoreType.DMA,
        ],
    )(inputs)


RPT_G = N_ROWS // NW


def _sc_gather_body(x16_hbm, t_hbm, xt_hbm, t_v, idx_v, gbuf, st, semg):
    c = lax.axis_index("c")
    s = lax.axis_index("s")
    base = c * ROWS_PER_CORE + s * RPT_G
    pltpu.sync_copy(t_hbm.at[pl.ds(base, RPT_G)], t_v)
    iota = lax.broadcasted_iota(jnp.int32, (L,), 0)

    dnums = lax.GatherDimensionNumbers(
        offset_dims=(), collapsed_slice_dims=(0,), start_index_map=(0,))
    for h in range(RPT_G // L):
        t16 = t_v[pl.ds(h * L, L)]
        rowvec = base + h * L + iota
        fi = rowvec * N_COLS + t16
        r128 = lax.shift_right_logical(fi, 7)
        lane7 = lax.bitwise_and(fi, 127)
        chunk = lax.shift_right_logical(lane7, 4)
        pos = lax.bitwise_and(lane7, 15)
        idx_v[...] = r128
        pltpu.async_copy(x16_hbm.at[idx_v], gbuf, semg).wait()
        w = jnp.zeros((L,), jnp.float32)
        for j in range(L):
            acc = jnp.zeros((L,), jnp.float32)
            for k in range(8):
                cjk = gbuf[j, pl.ds(k * L, L)]
                pk = lax.gather(cjk, pos[:, None], dnums, (1,),
                                mode=lax.GatherScatterMode.PROMISE_IN_BOUNDS)
                acc = jnp.where(chunk == k, pk, acc)
            w = jnp.where(iota == j, acc, w)
        st[...] = w
        pltpu.sync_copy(st, xt_hbm.at[pl.ds(base + h * L, L)])


def _sc_gather(x16, targets_i32):
    mesh = plsc.VectorSubcoreMesh(core_axis_name="c", subcore_axis_name="s")
    return pl.kernel(
        _sc_gather_body,
        out_type=jax.ShapeDtypeStruct((N_ROWS,), jnp.float32),
        mesh=mesh,
        scratch_types=[
            pltpu.VMEM((RPT_G,), jnp.int32),
            pltpu.VMEM((L,), jnp.int32),
            pltpu.VMEM((L, 128), jnp.float32),
            pltpu.VMEM((L,), jnp.float32),
            pltpu.SemaphoreType.DMA,
        ],
    )(x16, targets_i32)


def _combine_kernel(se_tc, sx_tc, se_sc, sx_sc, xt, out_ref):
    se = se_tc[...][:, 0] + jnp.sum(se_sc[...], axis=1)
    sx = sx_tc[...][:, 0] + jnp.sum(sx_sc[...], axis=1)
    losses = (jnp.log(se) - (SMOOTHING / N_COLS) * sx
              - CONFIDENCE * xt[...][:, 0])
    out_ref[...] = (jnp.sum(losses) * (1.0 / N_ROWS)).reshape(1, 1)


def _combine(se_tc, sx_tc, se_sc, sx_sc, xt2d):
    return pl.pallas_call(
        _combine_kernel,
        out_shape=jax.ShapeDtypeStruct((1, 1), jnp.float32),
    )(se_tc, sx_tc, se_sc, sx_sc, xt2d)


@functools.partial(jax.jit, static_argnames=())
def kernel(inputs, targets):
    x16 = inputs.reshape(-1, 128)
    t32 = targets.astype(jnp.int32)
    se_sc, sx_sc = _sc_main(inputs)
    xt = _sc_gather(x16, t32)
    se_tc, sx_tc = _tc_main(inputs)
    out = _combine(se_tc, sx_tc, se_sc, sx_sc, xt.reshape(N_ROWS, 1))
    return out.reshape(())


# speedup vs baseline: 2.1569x; 2.1569x over previous
import functools

import jax
import jax.numpy as jnp
from jax import lax
from jax.experimental import pallas as pl
from jax.experimental.pallas import tpu as pltpu
from jax.experimental.pallas import tpu_sc as plsc

N_ROWS = 1024
N_COLS = 100000
SMOOTHING = 0.1
CONFIDENCE = 1.0 - SMOOTHING

C_TC = 80000
K_SC = 19968
C_TAIL = C_TC + K_SC
TAIL_W = N_COLS - C_TAIL

R_BLK = 128
C_BLK = 16000

NC, NS, L = 2, 16, 16
ROWS_PER_CORE = N_ROWS // NC
N_CHUNKS = ROWS_PER_CORE // NS
FS = 4
RPF = NS // FS
U = 16


def _tc_kernel(x_ref, xtail_ref, t_ref, se_ref, sx_ref, xt_ref,
               s_acc, x_acc, t_acc):
    cb = pl.program_id(1)
    n_cb = pl.num_programs(1)
    t = t_ref[...]

    @pl.when(cb == 0)
    def _init():
        x = xtail_ref[...]
        lanes = jax.lax.broadcasted_iota(jnp.int32, x.shape, 1)
        cols = C_TAIL + lanes
        valid = lanes < TAIL_W
        e = jnp.exp(jnp.where(valid, jnp.minimum(x, 60.0), -jnp.inf))
        s_acc[...] = jnp.sum(e, axis=1, keepdims=True)
        x_acc[...] = jnp.sum(jnp.where(valid, x, 0.0), axis=1, keepdims=True)
        t_acc[...] = jnp.sum(jnp.where(cols == t, x, 0.0), axis=1,
                             keepdims=True)

    x = x_ref[...]
    e = jnp.exp(jnp.minimum(x, 60.0))
    s_acc[...] += jnp.sum(e, axis=1, keepdims=True)
    x_acc[...] += jnp.sum(x, axis=1, keepdims=True)
    col_ids = cb * C_BLK + jax.lax.broadcasted_iota(jnp.int32, x.shape, 1)
    t_acc[...] += jnp.sum(jnp.where(col_ids == t, x, 0.0), axis=1,
                          keepdims=True)

    @pl.when(cb == n_cb - 1)
    def _fin():
        se_ref[...] = s_acc[...]
        sx_ref[...] = x_acc[...]
        xt_ref[...] = t_acc[...]


def _tc_main(inputs, t2d):
    n_rb = N_ROWS // R_BLK
    n_cb = C_TC // C_BLK
    return pl.pallas_call(
        _tc_kernel,
        grid=(n_rb, n_cb),
        in_specs=[
            pl.BlockSpec((R_BLK, C_BLK), lambda rb, cb: (rb, cb)),
            pl.BlockSpec((R_BLK, 128), lambda rb, cb: (rb, C_TAIL // 128)),
            pl.BlockSpec((R_BLK, 1), lambda rb, cb: (rb, 0)),
        ],
        out_specs=[
            pl.BlockSpec((R_BLK, 1), lambda rb, cb: (rb, 0)),
            pl.BlockSpec((R_BLK, 1), lambda rb, cb: (rb, 0)),
            pl.BlockSpec((R_BLK, 1), lambda rb, cb: (rb, 0)),
        ],
        out_shape=[jax.ShapeDtypeStruct((N_ROWS, 1), jnp.float32),
                   jax.ShapeDtypeStruct((N_ROWS, 1), jnp.float32),
                   jax.ShapeDtypeStruct((N_ROWS, 1), jnp.float32)],
        scratch_shapes=[
            pltpu.VMEM((R_BLK, 1), jnp.float32),
            pltpu.VMEM((R_BLK, 1), jnp.float32),
            pltpu.VMEM((R_BLK, 1), jnp.float32),
        ],
    )(inputs, inputs, t2d)


def _tree(vs):
    while len(vs) > 1:
        nxt = [vs[j] + vs[j + 1] for j in range(0, len(vs) - 1, 2)]
        if len(vs) % 2:
            nxt.append(vs[-1])
        vs = nxt
    return vs[0]


_DNUMS = lax.GatherDimensionNumbers(
    offset_dims=(), collapsed_slice_dims=(0,), start_index_map=(0,))


def _sc_body(x_hbm, t_hbm, se_hbm, sx_hbm, xt_hbm,
             sp0, sp1, tbuf, tv, st_se, st_sx, st_xt, semf):
    c = lax.axis_index("c")
    s = lax.axis_index("s")
    zeros = jnp.zeros((L,), jnp.float32)
    iota = lax.broadcasted_iota(jnp.int32, (L,), 0)
    svec = jnp.full((L,), s, jnp.int32)
    sps = (sp0, sp1)

    pltpu.sync_copy(t_hbm.at[pl.ds(c * ROWS_PER_CORE, ROWS_PER_CORE)], tv)

    def fill_desc(g, spbuf):
        row0 = c * ROWS_PER_CORE + g * NS + s * RPF
        return pltpu.make_async_copy(
            x_hbm.at[pl.ds(row0, RPF), pl.ds(C_TC, K_SC)],
            spbuf.at[pl.ds(s * RPF, RPF)], semf)

    @pl.when(s < FS)
    def _prime():
        fill_desc(0, sp0).start()
        fill_desc(0, sp0).wait()

    plsc.subcore_barrier()

    for g in range(N_CHUNKS):
        cur = sps[g % 2]
        nxt = sps[(g + 1) % 2]
        if g + 1 < N_CHUNKS:
            @pl.when(s < FS)
            def _start_next(g=g, nxt=nxt):
                fill_desc(g + 1, nxt).start()

        pltpu.sync_copy(cur.at[s], tbuf)

        t16 = tv[pl.ds(g * NS, NS)]
        tloc = lax.gather(t16, svec[:, None], _DNUMS, (1,),
                          mode=lax.GatherScatterMode.PROMISE_IN_BOUNDS) - C_TC

        def ch(i, cry):
            se, sx, xt = cry
            base = i * (U * L)
            for u in range(U):
                xv = tbuf[pl.ds(base + u * L, L)]
                colv = base + u * L + iota
                if u == 0:
                    es = [jnp.exp(jnp.minimum(xv, 60.0))]
                    xs = [xv]
                    ms = [jnp.where(colv == tloc, xv, 0.0)]
                else:
                    es.append(jnp.exp(jnp.minimum(xv, 60.0)))
                    xs.append(xv)
                    ms.append(jnp.where(colv == tloc, xv, 0.0))
            return se + _tree(es), sx + _tree(xs), xt + _tree(ms)

        se16, sx16, xt16 = lax.fori_loop(0, K_SC // (U * L), ch,
                                         (zeros, zeros, zeros))
        st_se[...] = se16
        st_sx[...] = sx16
        st_xt[...] = xt16
        row = c * ROWS_PER_CORE + g * NS + s
        pltpu.sync_copy(st_se, se_hbm.at[row])
        pltpu.sync_copy(st_sx, sx_hbm.at[row])
        pltpu.sync_copy(st_xt, xt_hbm.at[row])

        if g + 1 < N_CHUNKS:
            @pl.when(s < FS)
            def _wait_next(g=g, nxt=nxt):
                fill_desc(g + 1, nxt).wait()

        plsc.subcore_barrier()


def _sc_main(inputs, t32):
    mesh = plsc.VectorSubcoreMesh(core_axis_name="c", subcore_axis_name="s")
    return pl.kernel(
        _sc_body,
        out_type=(jax.ShapeDtypeStruct((N_ROWS, L), jnp.float32),
                  jax.ShapeDtypeStruct((N_ROWS, L), jnp.float32),
                  jax.ShapeDtypeStruct((N_ROWS, L), jnp.float32)),
        mesh=mesh,
        scratch_types=[
            pltpu.MemorySpace.VMEM_SHARED((NS, K_SC), jnp.float32),
            pltpu.MemorySpace.VMEM_SHARED((NS, K_SC), jnp.float32),
            pltpu.VMEM((K_SC,), jnp.float32),
            pltpu.VMEM((ROWS_PER_CORE,), jnp.int32),
            pltpu.VMEM((L,), jnp.float32),
            pltpu.VMEM((L,), jnp.float32),
            pltpu.VMEM((L,), jnp.float32),
            pltpu.SemaphoreType.DMA,
        ],
    )(inputs, t32)


def _combine_kernel(se_tc, sx_tc, xt_tc, se_sc, sx_sc, xt_sc, out_ref):
    se = se_tc[...][:, 0] + jnp.sum(se_sc[...], axis=1)
    sx = sx_tc[...][:, 0] + jnp.sum(sx_sc[...], axis=1)
    xt = xt_tc[...][:, 0] + jnp.sum(xt_sc[...], axis=1)
    losses = (jnp.log(se) - (SMOOTHING / N_COLS) * sx - CONFIDENCE * xt)
    out_ref[...] = (jnp.sum(losses) * (1.0 / N_ROWS)).reshape(1, 1)


def _combine(se_tc, sx_tc, xt_tc, se_sc, sx_sc, xt_sc):
    return pl.pallas_call(
        _combine_kernel,
        out_shape=jax.ShapeDtypeStruct((1, 1), jnp.float32),
    )(se_tc, sx_tc, xt_tc, se_sc, sx_sc, xt_sc)


@functools.partial(jax.jit, static_argnames=())
def kernel(inputs, targets):
    t32 = targets.astype(jnp.int32)
    t2d = t32.reshape(N_ROWS, 1)
    se_sc, sx_sc, xt_sc = _sc_main(inputs, t32)
    se_tc, sx_tc, xt_tc = _tc_main(inputs, t2d)
    out = _combine(se_tc, sx_tc, xt_tc, se_sc, sx_sc, xt_sc)
    return out.reshape(())
